# Initial kernel scaffold; baseline (speedup 1.0000x reference)
import jax
import jax.numpy as jnp
from jax.experimental import pallas as pl


def kernel(g):
    raise NotImplementedError("write your pallas kernel here")


# speedup vs baseline: 1.2629x; 1.2629x over previous
import functools

import jax
import jax.numpy as jnp
from jax import lax
from jax.experimental import pallas as pl
from jax.experimental.pallas import tpu as pltpu
from jax.experimental.pallas import tpu_sc as plsc

B, T, H = 16, 4096, 32
BT = 64
NC, NS, L = 2, 16, 16

TOTAL = B * T * H
N_CHUNKS = (B * T) // BT
CHUNK_ELEMS = BT * H
NW = NC * NS
CHUNKS_PER_W = N_CHUNKS // NW
BLK_CHUNKS = 8
BLK_ELEMS = BLK_CHUNKS * CHUNK_ELEMS
N_BLKS = CHUNKS_PER_W // BLK_CHUNKS
SPAN = CHUNKS_PER_W * CHUNK_ELEMS


def _body(g_hbm, out_hbm, buf):
    wid = lax.axis_index("s") * NC + lax.axis_index("c")
    wbase = wid * SPAN
    zero = jnp.zeros((L,), jnp.float32)

    def blk_body(blk, _):
        hbase = wbase + blk * BLK_ELEMS
        pltpu.sync_copy(g_hbm.at[pl.ds(hbase, BLK_ELEMS)], buf)

        def chunk_body(c, _):
            cbase = c * CHUNK_ELEMS

            def t_body(t, accs):
                a0, a1 = accs
                off = cbase + t * H
                a0 = a0 + buf[pl.ds(off, L)]
                a1 = a1 + buf[pl.ds(off + L, L)]
                buf[pl.ds(off, L)] = a0
                buf[pl.ds(off + L, L)] = a1
                return (a0, a1)

            lax.fori_loop(0, BT, t_body, (zero, zero), unroll=4)
            return 0

        lax.fori_loop(0, BLK_CHUNKS, chunk_body, 0)
        pltpu.sync_copy(buf, out_hbm.at[pl.ds(hbase, BLK_ELEMS)])
        return 0

    lax.fori_loop(0, N_BLKS, blk_body, 0)


@jax.jit
def kernel(g):
    gf = g.reshape(TOTAL)
    run = pl.kernel(
        _body,
        out_type=jax.ShapeDtypeStruct((TOTAL,), jnp.float32),
        mesh=plsc.VectorSubcoreMesh(
            core_axis_name="c", subcore_axis_name="s",
            num_cores=NC, num_subcores=NS,
        ),
        scratch_types=[pltpu.VMEM((BLK_ELEMS,), jnp.float32)],
    )
    return run(gf).reshape(B, T, H)


# speedup vs baseline: 1.4166x; 1.1217x over previous
import jax
import jax.numpy as jnp
from jax import lax
from jax.experimental import pallas as pl
from jax.experimental.pallas import tpu as pltpu
from jax.experimental.pallas import tpu_sc as plsc

B, T, H = 16, 4096, 32
BT = 64
NC, NS, L = 2, 16, 16

NW = NC * NS
T_PER_W = (B * T) // NW
BLK_T = 512
BLK_CHUNKS = BLK_T // BT
N_BLKS = T_PER_W // BLK_T


def _body(g_hbm, out_hbm, buf):
    wid = lax.axis_index("s") * NC + lax.axis_index("c")
    b = wid // 2
    t0 = (wid % 2) * T_PER_W
    zero = jnp.zeros((L,), jnp.float32)

    def blk_body(blk, _):
        ts = t0 + blk * BLK_T
        pltpu.sync_copy(g_hbm.at[b, pl.ds(ts, BLK_T), :], buf)

        def chunk_body(c, _):
            cbase = c * BT

            def t_body(t, accs):
                a0, a1 = accs
                row = cbase + t
                a0 = a0 + buf[row, pl.ds(0, L)]
                a1 = a1 + buf[row, pl.ds(L, L)]
                buf[row, pl.ds(0, L)] = a0
                buf[row, pl.ds(L, L)] = a1
                return (a0, a1)

            lax.fori_loop(0, BT, t_body, (zero, zero), unroll=4)
            return 0

        lax.fori_loop(0, BLK_CHUNKS, chunk_body, 0)
        pltpu.sync_copy(buf, out_hbm.at[b, pl.ds(ts, BLK_T), :])
        return 0

    lax.fori_loop(0, N_BLKS, blk_body, 0)


@jax.jit
def kernel(g):
    run = pl.kernel(
        _body,
        out_type=jax.ShapeDtypeStruct((B, T, H), jnp.float32),
        mesh=plsc.VectorSubcoreMesh(
            core_axis_name="c", subcore_axis_name="s",
            num_cores=NC, num_subcores=NS,
        ),
        scratch_types=[pltpu.VMEM((BLK_T, H), jnp.float32)],
    )
    return run(g)


# speedup vs baseline: 4.0456x; 2.8558x over previous
import jax
import jax.numpy as jnp
from jax import lax
from jax.experimental import pallas as pl
from jax.experimental.pallas import tpu as pltpu
from jax.experimental.pallas import tpu_sc as plsc

B, T, H = 16, 4096, 32
BT = 64
NC, NS, L = 2, 16, 16

HT = H // 8
TT = T // 128
NW = NC * NS
N_PAIRS = B * HT
VPC = BT // L


def _body(x_hbm, out_hbm, buf):
    wid = lax.axis_index("s") * NC + lax.axis_index("c")

    def pair_body(p, _):
        q = wid * 2 + p
        b = q // HT
        ht = q % HT
        pltpu.sync_copy(x_hbm.at[b, ht], buf)

        def tt_body(tt, _):
            for h8 in range(8):
                for chunk in range(2):
                    base = chunk * BT
                    car = jnp.float32(0.0)
                    for k in range(VPC):
                        off = base + k * L
                        v = buf[tt, h8, pl.ds(off, L)]
                        s = plsc.cumsum(v) + car
                        buf[tt, h8, pl.ds(off, L)] = s
                        car = car + jnp.sum(v)
            return 0

        lax.fori_loop(0, TT, tt_body, 0)
        pltpu.sync_copy(buf, out_hbm.at[b, ht])
        return 0

    lax.fori_loop(0, 2, pair_body, 0)


@jax.jit
def kernel(g):
    x = g.transpose(0, 2, 1).reshape(B, HT, 8, TT, 128).transpose(0, 1, 3, 2, 4)
    run = pl.kernel(
        _body,
        out_type=jax.ShapeDtypeStruct((B, HT, TT, 8, 128), jnp.float32),
        mesh=plsc.VectorSubcoreMesh(
            core_axis_name="c", subcore_axis_name="s",
            num_cores=NC, num_subcores=NS,
        ),
        scratch_types=[pltpu.VMEM((TT, 8, 128), jnp.float32)],
        compiler_params=pltpu.CompilerParams(needs_layout_passes=False),
    )
    y = run(x)
    return y.transpose(0, 1, 3, 2, 4).reshape(B, H, T).transpose(0, 2, 1)


# speedup vs baseline: 4.3278x; 1.0697x over previous
import jax
import jax.numpy as jnp
from jax import lax
from jax.experimental import pallas as pl
from jax.experimental.pallas import tpu as pltpu
from jax.experimental.pallas import tpu_sc as plsc

B, T, H = 16, 4096, 32
BT = 64
NC, NS, L = 2, 16, 16

HT = H // 8
TT = T // 128
NW = NC * NS
VPC = BT // L


def _compute(buf):
    def tt_body(tt, _):
        for h8 in range(8):
            for chunk in range(2):
                base = chunk * BT
                car = jnp.float32(0.0)
                for k in range(VPC):
                    off = base + k * L
                    v = buf[tt, h8, pl.ds(off, L)]
                    s = plsc.cumsum(v) + car
                    buf[tt, h8, pl.ds(off, L)] = s
                    car = car + jnp.sum(v)
        return 0

    lax.fori_loop(0, TT, tt_body, 0)


def _body(x_hbm, out_hbm, buf0, buf1, si0, si1, so0, so1):
    wid = lax.axis_index("s") * NC + lax.axis_index("c")
    q0 = wid * 2
    b0, ht0 = q0 // HT, q0 % HT
    b1, ht1 = (q0 + 1) // HT, (q0 + 1) % HT

    d_in0 = pltpu.async_copy(x_hbm.at[b0, ht0], buf0, si0)
    d_in1 = pltpu.async_copy(x_hbm.at[b1, ht1], buf1, si1)
    d_in0.wait()
    _compute(buf0)
    d_out0 = pltpu.async_copy(buf0, out_hbm.at[b0, ht0], so0)
    d_in1.wait()
    _compute(buf1)
    d_out1 = pltpu.async_copy(buf1, out_hbm.at[b1, ht1], so1)
    d_out0.wait()
    d_out1.wait()


@jax.jit
def kernel(g):
    x = g.transpose(0, 2, 1).reshape(B, HT, 8, TT, 128).transpose(0, 1, 3, 2, 4)
    run = pl.kernel(
        _body,
        out_type=jax.ShapeDtypeStruct((B, HT, TT, 8, 128), jnp.float32),
        mesh=plsc.VectorSubcoreMesh(
            core_axis_name="c", subcore_axis_name="s",
            num_cores=NC, num_subcores=NS,
        ),
        scratch_types=[
            pltpu.VMEM((TT, 8, 128), jnp.float32),
            pltpu.VMEM((TT, 8, 128), jnp.float32),
            pltpu.SemaphoreType.DMA,
            pltpu.SemaphoreType.DMA,
            pltpu.SemaphoreType.DMA,
            pltpu.SemaphoreType.DMA,
        ],
        compiler_params=pltpu.CompilerParams(needs_layout_passes=False),
    )
    y = run(x)
    return y.transpose(0, 1, 3, 2, 4).reshape(B, H, T).transpose(0, 2, 1)


# speedup vs baseline: 4.4244x; 1.0223x over previous
import jax
import jax.numpy as jnp
from jax import lax
from jax.experimental import pallas as pl
from jax.experimental.pallas import tpu as pltpu
from jax.experimental.pallas import tpu_sc as plsc

B, T, H = 16, 4096, 32
BT = 64
NC, NS, L = 2, 16, 16

HT = H // 8
TT = T // 128
NW = NC * NS
VPC = BT // L


def _compute(buf):
    def tt_body(tt, _):
        for h8 in range(8):
            for chunk in range(2):
                base = chunk * BT
                car = jnp.float32(0.0)
                for k in range(VPC):
                    off = base + k * L
                    v = buf[tt, h8, pl.ds(off, L)]
                    s = plsc.cumsum(v) + car
                    buf[tt, h8, pl.ds(off, L)] = s
                    car = jnp.squeeze(lax.slice(s, (15,), (16,)))
        return 0

    lax.fori_loop(0, TT, tt_body, 0)


def _body(x_hbm, out_hbm, buf0, buf1, si0, si1, so0, so1):
    wid = lax.axis_index("s") * NC + lax.axis_index("c")
    q0 = wid * 2
    b0, ht0 = q0 // HT, q0 % HT
    b1, ht1 = (q0 + 1) // HT, (q0 + 1) % HT

    d_in0 = pltpu.async_copy(x_hbm.at[b0, ht0], buf0, si0)
    d_in1 = pltpu.async_copy(x_hbm.at[b1, ht1], buf1, si1)
    d_in0.wait()
    _compute(buf0)
    d_out0 = pltpu.async_copy(buf0, out_hbm.at[b0, ht0], so0)
    d_in1.wait()
    _compute(buf1)
    d_out1 = pltpu.async_copy(buf1, out_hbm.at[b1, ht1], so1)
    d_out0.wait()
    d_out1.wait()


@jax.jit
def kernel(g):
    x = g.transpose(0, 2, 1).reshape(B, HT, 8, TT, 128).transpose(0, 1, 3, 2, 4)
    run = pl.kernel(
        _body,
        out_type=jax.ShapeDtypeStruct((B, HT, TT, 8, 128), jnp.float32),
        mesh=plsc.VectorSubcoreMesh(
            core_axis_name="c", subcore_axis_name="s",
            num_cores=NC, num_subcores=NS,
        ),
        scratch_types=[
            pltpu.VMEM((TT, 8, 128), jnp.float32),
            pltpu.VMEM((TT, 8, 128), jnp.float32),
            pltpu.SemaphoreType.DMA,
            pltpu.SemaphoreType.DMA,
            pltpu.SemaphoreType.DMA,
            pltpu.SemaphoreType.DMA,
        ],
        compiler_params=pltpu.CompilerParams(needs_layout_passes=False),
    )
    y = run(x)
    return y.transpose(0, 1, 3, 2, 4).reshape(B, H, T).transpose(0, 2, 1)
